# baseline (speedup 1.0000x reference)
import jax
import jax.numpy as jnp
from jax import lax
from jax.experimental import pallas as pl
from jax.experimental.pallas import tpu as pltpu

_DIM = 32768
_K = 4096
_BATCH = 128
_ROW_BLK = 32


def _body(w_ref, x_ref, o_ref, mask_ref):
    @pl.when(pl.program_id(0) == 0)
    def _compute_mask():
        w = w_ref[...]
        bits = lax.bitcast_convert_type(w, jnp.uint32)
        neg = bits >= jnp.uint32(0x80000000)
        key = jnp.where(neg, ~bits, bits | jnp.uint32(0x80000000))

        jvec = lax.broadcasted_iota(jnp.uint32, (16, 1), 0)

        def step(i, t):
            b = jnp.uint32(28) - jnp.uint32(4) * i.astype(jnp.uint32)
            cands = t | jnp.left_shift(jvec, b)
            cnts = jnp.sum((key >= cands).astype(jnp.int32), axis=1,
                           keepdims=True)
            j_star = (jnp.sum((cnts >= _K).astype(jnp.int32)) - 1).astype(jnp.uint32)
            return t | jnp.left_shift(j_star, b)

        t = lax.fori_loop(0, 8, step, jnp.uint32(0))
        mask_ref[...] = (key >= t).astype(jnp.float32)

    o_ref[...] = x_ref[...] * mask_ref[...]


def kernel(inputs, w):
    w2 = w.reshape(1, _DIM)
    return pl.pallas_call(
        _body,
        grid=(_BATCH // _ROW_BLK,),
        in_specs=[
            pl.BlockSpec((1, _DIM), lambda i: (0, 0)),
            pl.BlockSpec((_ROW_BLK, _DIM), lambda i: (i, 0)),
        ],
        out_specs=pl.BlockSpec((_ROW_BLK, _DIM), lambda i: (i, 0)),
        out_shape=jax.ShapeDtypeStruct((_BATCH, _DIM), jnp.float32),
        scratch_shapes=[pltpu.VMEM((1, _DIM), jnp.float32)],
    )(w2, inputs)


# speedup vs baseline: 1.1708x; 1.1708x over previous
import jax
import jax.numpy as jnp
from jax import lax
from jax.experimental import pallas as pl
from jax.experimental.pallas import tpu as pltpu

_DIM = 32768
_K = 4096
_BATCH = 128
_ROW_BLK = 64


def _body(w_ref, x_ref, o_ref, mask_ref):
    @pl.when(pl.program_id(0) == 0)
    def _compute_mask():
        w = w_ref[...]
        bits = lax.bitcast_convert_type(w, jnp.uint32)
        neg = bits >= jnp.uint32(0x80000000)
        key = jnp.where(neg, ~bits, bits | jnp.uint32(0x80000000))

        jvec = lax.broadcasted_iota(jnp.uint32, (16, 1), 0)

        def step(i, t):
            b = jnp.uint32(28) - jnp.uint32(4) * i.astype(jnp.uint32)
            cands = t | jnp.left_shift(jvec, b)
            cnts = jnp.sum((key >= cands).astype(jnp.int32), axis=1,
                           keepdims=True)
            j_star = jnp.sum((cnts >= _K).astype(jnp.int32), axis=0,
                             keepdims=True) - 1
            return t | jnp.left_shift(
                jnp.broadcast_to(j_star.astype(jnp.uint32), (16, 1)), b)

        t = lax.fori_loop(0, 8, step, jnp.zeros((16, 1), jnp.uint32))
        mask_ref[...] = (key >= t[0:1, 0:1]).astype(jnp.float32)

    o_ref[...] = x_ref[...] * mask_ref[...]


def kernel(inputs, w):
    w2 = w.reshape(1, _DIM)
    return pl.pallas_call(
        _body,
        grid=(_BATCH // _ROW_BLK,),
        in_specs=[
            pl.BlockSpec((1, _DIM), lambda i: (0, 0)),
            pl.BlockSpec((_ROW_BLK, _DIM), lambda i: (i, 0)),
        ],
        out_specs=pl.BlockSpec((_ROW_BLK, _DIM), lambda i: (i, 0)),
        out_shape=jax.ShapeDtypeStruct((_BATCH, _DIM), jnp.float32),
        scratch_shapes=[pltpu.VMEM((1, _DIM), jnp.float32)],
    )(w2, inputs)
